# baseline (speedup 1.0000x reference)
import jax
import jax.numpy as jnp
from jax.experimental import pallas as pl


def _mm_bf16(a, b):
    return jax.lax.dot_general(a, b, (((1,), (0,)), ((), ())),
                               preferred_element_type=jnp.float32)


def _matmul3(h, w_hi, w_lo):
    h1 = h.astype(jnp.bfloat16)
    h2 = (h - h1.astype(jnp.float32)).astype(jnp.bfloat16)
    return _mm_bf16(h1, w_hi) + (_mm_bf16(h1, w_lo) + _mm_bf16(h2, w_hi))


def _matmul_ta_bf16(a, b):
    return jax.lax.dot_general(a, b, (((0,), (0,)), ((), ())),
                               preferred_element_type=jnp.float32)


def _body(x_ref, M_ref, *refs):
    w = [(refs[2 * i][...], refs[2 * i + 1][...]) for i in range(9)]
    b = [refs[18 + i][...] for i in range(9)]
    out_ref = refs[27]
    (W1, W2, W3, Wg1, Wg2, Wg3, Wg4, Wg5, Wg6) = w
    (b1, b2, b3, bg1, bg2, bg3, bg4, bg5, bg6) = b

    n = M_ref.shape[0]
    M = M_ref[...]
    ones = jnp.ones((n, 1), jnp.bfloat16)
    deg = _matmul_ta_bf16(M, ones) + 1.0
    dinv = jax.lax.rsqrt(deg)
    dinv2 = dinv * dinv

    def gcn(h, W, bb):
        hw = _matmul3(h, *W)
        t = _matmul_ta_bf16(M, (hw * dinv).astype(jnp.bfloat16))
        return t * dinv + hw * dinv2 + bb

    x = x_ref[...]
    x1l = _matmul3(x, *W1) + b1
    x1 = x1l + jax.nn.relu(gcn(x1l, Wg1, bg1))
    x2l = _matmul3(x1, *W2) + b2
    x2 = x2l + jax.nn.relu(gcn(x2l, Wg2, bg2))
    x3l = _matmul3(x2, *W3) + b3
    x3 = x3l + 0.5 * jax.nn.relu(gcn(x3l, Wg3, bg3))
    x4 = x3 + 0.5 * jax.nn.relu(gcn(x3, Wg4, bg4))
    x5 = x4 + 0.25 * jax.nn.relu(gcn(x4, Wg5, bg5))
    out_ref[...] = x5 + 0.25 * gcn(x5, Wg6, bg6)


def kernel(x, A_neg, A_pos, W1, b1, W2, b2, W3, b3, Wg1, bg1, Wg2, bg2,
           Wg3, bg3, Wg4, bg4, Wg5, bg5, Wg6, bg6):
    del A_pos
    n, dout = x.shape[0], Wg3.shape[0]

    Mbf = (A_neg != 0).astype(jnp.bfloat16)

    d2 = W2.shape[1]
    pad = dout - d2
    W2p = jnp.pad(W2, ((0, 0), (0, pad)))
    b2p = jnp.pad(b2, (0, pad))
    Wg2p = jnp.pad(Wg2, ((0, pad), (0, pad)))
    bg2p = jnp.pad(bg2, (0, pad))
    W3p = jnp.pad(W3, ((0, pad), (0, 0)))

    def split(wm):
        hi = wm.astype(jnp.bfloat16)
        lo = (wm - hi.astype(jnp.float32)).astype(jnp.bfloat16)
        return hi, lo

    weights = [W1, W2p, W3p, Wg1, Wg2p, Wg3, Wg4, Wg5, Wg6]
    wrefs = []
    for wm in weights:
        wrefs.extend(split(wm))
    biases = [b1, b2p, b3, bg1, bg2p, bg3, bg4, bg5, bg6]
    brefs = [v.reshape(1, -1) for v in biases]

    return pl.pallas_call(
        _body,
        out_shape=jax.ShapeDtypeStruct((n, dout), jnp.float32),
    )(x, Mbf, *wrefs, *brefs)


# speedup vs baseline: 1.7860x; 1.7860x over previous
import jax
import jax.numpy as jnp
from jax.experimental import pallas as pl


def _mm_bf16(a, b):
    return jax.lax.dot_general(a, b, (((1,), (0,)), ((), ())),
                               preferred_element_type=jnp.float32)


def _matmul_ta_bf16(a, b):
    return jax.lax.dot_general(a, b, (((0,), (0,)), ((), ())),
                               preferred_element_type=jnp.float32)


def _split(v):
    hi = v.astype(jnp.bfloat16)
    lo = (v - hi.astype(jnp.float32)).astype(jnp.bfloat16)
    return hi, lo


def _matmul3(h, w):
    h1, h2 = _split(h)
    w1, w2 = w
    return _mm_bf16(h1, w1) + (_mm_bf16(h1, w2) + _mm_bf16(h2, w1))


def _body(x_ref, A_ref, W1_ref, b1_ref, W2_ref, b2_ref, W3_ref, b3_ref,
          Wg1_ref, bg1_ref, Wg2_ref, bg2_ref, Wg3_ref, bg3_ref,
          Wg4_ref, bg4_ref, Wg5_ref, bg5_ref, Wg6_ref, bg6_ref, out_ref):
    n = A_ref.shape[0]
    M = (A_ref[...] != 0).astype(jnp.bfloat16)
    ones = jnp.ones((n, 1), jnp.bfloat16)
    deg = _matmul_ta_bf16(M, ones) + 1.0
    dinv = jax.lax.rsqrt(deg)
    dinv2 = dinv * dinv

    W1 = _split(W1_ref[...])
    W2 = _split(W2_ref[...])
    W3 = _split(W3_ref[...])
    Wg1 = _split(Wg1_ref[...])
    Wg2 = _split(Wg2_ref[...])
    Wg3 = _split(Wg3_ref[...])
    Wg4 = _split(Wg4_ref[...])
    Wg5 = _split(Wg5_ref[...])
    Wg6 = _split(Wg6_ref[...])

    def gcn(h, w, bb):
        hw = _matmul3(h, w)
        t = _matmul_ta_bf16(M, (hw * dinv).astype(jnp.bfloat16))
        return t * dinv + hw * dinv2 + bb

    x = x_ref[...]
    x1l = _matmul3(x, W1) + b1_ref[...]
    x1 = x1l + jax.nn.relu(gcn(x1l, Wg1, bg1_ref[...]))
    x2l = _matmul3(x1, W2) + b2_ref[...]
    x2 = x2l + jax.nn.relu(gcn(x2l, Wg2, bg2_ref[...]))
    x3l = _matmul3(x2, W3) + b3_ref[...]
    x3 = x3l + 0.5 * jax.nn.relu(gcn(x3l, Wg3, bg3_ref[...]))
    x4 = x3 + 0.5 * jax.nn.relu(gcn(x3, Wg4, bg4_ref[...]))
    x5 = x4 + 0.25 * jax.nn.relu(gcn(x4, Wg5, bg5_ref[...]))
    out_ref[...] = x5 + 0.25 * gcn(x5, Wg6, bg6_ref[...])


def kernel(x, A_neg, A_pos, W1, b1, W2, b2, W3, b3, Wg1, bg1, Wg2, bg2,
           Wg3, bg3, Wg4, bg4, Wg5, bg5, Wg6, bg6):
    del A_pos
    n, dout = x.shape[0], Wg3.shape[0]
    row = lambda v: v.reshape(1, -1)
    return pl.pallas_call(
        _body,
        out_shape=jax.ShapeDtypeStruct((n, dout), jnp.float32),
    )(x, A_neg, W1, row(b1), W2, row(b2), W3, row(b3),
      Wg1, row(bg1), Wg2, row(bg2), Wg3, row(bg3),
      Wg4, row(bg4), Wg5, row(bg5), Wg6, row(bg6))
